# baseline (speedup 1.0000x reference)
import jax
import jax.numpy as jnp
from jax.experimental import pallas as pl
from jax.experimental.pallas import tpu as pltpu

_K = 32
_ALPHA = 0.2
_NEG = -9e15
_ROW_BLK = 1024
_ONE_BITS = 0x3F800001


def _wh_kernel(h_ref, w_ref, wh_ref):
    wh_ref[0] = jnp.dot(h_ref[0], w_ref[...], preferred_element_type=jnp.float32)


def _gat_kernel(wh_ref, adjT_ref, a_ref, out_ref, x_ref):
    f_out = wh_ref.shape[2]
    r = pl.program_id(1)
    wh = wh_ref[0]
    adjT = adjT_ref[0]
    a1 = a_ref[0:1, 0:f_out]
    a2 = a_ref[0:1, f_out:2 * f_out]
    wh_rows = wh_ref[0, pl.ds(r * _ROW_BLK, _ROW_BLK), :]
    f1c = jax.lax.dot_general(wh_rows, a1, (((1,), (1,)), ((), ())),
                              preferred_element_type=jnp.float32)
    f2r = jax.lax.dot_general(a2, wh, (((1,), (1,)), ((), ())),
                              preferred_element_type=jnp.float32)
    f1r = jnp.transpose(f1c)
    f2c = jnp.transpose(f2r)
    e = f2c + f1r
    e = jnp.where(e >= 0, e, _ALPHA * e)
    e = jnp.where(adjT > 0, e, _NEG)
    m = jnp.max(e, axis=0, keepdims=True)
    p = jnp.exp(e - m)
    s = jnp.sum(p, axis=0, keepdims=True)
    attn = p / s
    x_ref[...] = attn

    cols = attn.shape[1]
    lo0 = jnp.zeros((1, cols), jnp.int32)
    hi0 = jnp.full((1, cols), _ONE_BITS, jnp.int32)

    def cond(state):
        lo, hi, it = state
        return jnp.logical_and(it < 32, jnp.any(hi - lo > 1))

    def body(state):
        lo, hi, it = state
        mid = jax.lax.shift_right_logical(lo + hi, 1)
        t = jax.lax.bitcast_convert_type(mid, jnp.float32)
        cf = jnp.sum(jnp.where(x_ref[...] >= t, 1.0, 0.0), axis=0,
                     keepdims=True)
        gek = cf >= _K
        eqk = cf == _K
        lo = jnp.where(gek, mid, lo)
        hi = jnp.where(eqk, mid + 1, jnp.where(gek, hi, mid))
        return lo, hi, it + 1

    lo, _, _ = jax.lax.while_loop(cond, body, (lo0, hi0, jnp.int32(0)))
    thresh = jax.lax.bitcast_convert_type(lo, jnp.float32)

    x = x_ref[...]
    xm = jnp.where(x < thresh, 0.0, x)
    hp = jax.lax.dot_general(xm, wh, (((0,), (0,)), ((), ())),
                             preferred_element_type=jnp.float32)
    out_ref[0] = jnp.where(hp > 0, hp, jnp.exp(hp) - 1.0)


def kernel(h, adj, W, a):
    b, n, f_in = h.shape
    f_out = W.shape[1]
    wh = pl.pallas_call(
        _wh_kernel,
        grid=(b,),
        in_specs=[
            pl.BlockSpec((1, n, f_in), lambda i: (i, 0, 0)),
            pl.BlockSpec((f_in, f_out), lambda i: (0, 0)),
        ],
        out_specs=pl.BlockSpec((1, n, f_out), lambda i: (i, 0, 0)),
        out_shape=jax.ShapeDtypeStruct((b, n, f_out), jnp.float32),
        compiler_params=pltpu.CompilerParams(
            dimension_semantics=("parallel",)),
    )(h, W)

    a_row = a.reshape(1, 2 * f_out)
    adjT = jnp.swapaxes(adj, 1, 2)
    out = pl.pallas_call(
        _gat_kernel,
        grid=(b, n // _ROW_BLK),
        in_specs=[
            pl.BlockSpec((1, n, f_out), lambda i, j: (i, 0, 0)),
            pl.BlockSpec((1, n, _ROW_BLK), lambda i, j: (i, 0, j)),
            pl.BlockSpec((1, 2 * f_out), lambda i, j: (0, 0)),
        ],
        out_specs=pl.BlockSpec((1, _ROW_BLK, f_out), lambda i, j: (i, j, 0)),
        out_shape=jax.ShapeDtypeStruct((b, n, f_out), jnp.float32),
        scratch_shapes=[
            pltpu.VMEM((n, _ROW_BLK), jnp.float32),
        ],
        compiler_params=pltpu.CompilerParams(
            dimension_semantics=("parallel", "parallel")),
    )(wh, adjT, a_row)
    return out


# speedup vs baseline: 1.0410x; 1.0410x over previous
import jax
import jax.numpy as jnp
from jax.experimental import pallas as pl
from jax.experimental.pallas import tpu as pltpu

_K = 32
_ALPHA = 0.2
_NEG = -9e15
_ROW_BLK = 512
_ONE_BITS = 0x3F800001


def _wh_kernel(h_ref, w_ref, wh_ref):
    wh_ref[0] = jnp.dot(h_ref[0], w_ref[...], preferred_element_type=jnp.float32)


def _gat_kernel(wh_ref, adjT_ref, a_ref, out_ref, x_ref):
    f_out = wh_ref.shape[2]
    r = pl.program_id(1)
    wh = wh_ref[0]
    adjT = adjT_ref[0]
    a1 = a_ref[0:1, 0:f_out]
    a2 = a_ref[0:1, f_out:2 * f_out]
    wh_rows = wh_ref[0, pl.ds(r * _ROW_BLK, _ROW_BLK), :]
    f1c = jax.lax.dot_general(wh_rows, a1, (((1,), (1,)), ((), ())),
                              preferred_element_type=jnp.float32)
    f2r = jax.lax.dot_general(a2, wh, (((1,), (1,)), ((), ())),
                              preferred_element_type=jnp.float32)
    f1r = jnp.transpose(f1c)
    f2c = jnp.transpose(f2r)
    e = f2c + f1r
    e = jnp.where(e >= 0, e, _ALPHA * e)
    e = jnp.where(adjT > 0, e, _NEG)
    m = jnp.max(e, axis=0, keepdims=True)
    p = jnp.exp(e - m)
    s = jnp.sum(p, axis=0, keepdims=True)
    attn = p / s
    x_ref[...] = attn

    cols = attn.shape[1]
    lo0 = jnp.zeros((1, cols), jnp.int32)
    hi0 = jnp.full((1, cols), _ONE_BITS, jnp.int32)

    def body(it, state):
        lo, hi = state
        mid = jax.lax.shift_right_logical(lo + hi, 1)
        t = jax.lax.bitcast_convert_type(mid, jnp.float32)
        cf = jnp.sum(jnp.where(x_ref[...] >= t, 1.0, 0.0), axis=0,
                     keepdims=True)
        gek = cf >= _K
        eqk = cf == _K
        lo = jnp.where(gek, mid, lo)
        hi = jnp.where(eqk, mid + 1, jnp.where(gek, hi, mid))
        return lo, hi

    lo, _ = jax.lax.fori_loop(0, 30, body, (lo0, hi0))
    thresh = jax.lax.bitcast_convert_type(lo, jnp.float32)

    x = x_ref[...]
    xm = jnp.where(x < thresh, 0.0, x)
    hp = jax.lax.dot_general(xm, wh, (((0,), (0,)), ((), ())),
                             preferred_element_type=jnp.float32)
    out_ref[0] = jnp.where(hp > 0, hp, jnp.exp(hp) - 1.0)


def kernel(h, adj, W, a):
    b, n, f_in = h.shape
    f_out = W.shape[1]
    wh = pl.pallas_call(
        _wh_kernel,
        grid=(b,),
        in_specs=[
            pl.BlockSpec((1, n, f_in), lambda i: (i, 0, 0)),
            pl.BlockSpec((f_in, f_out), lambda i: (0, 0)),
        ],
        out_specs=pl.BlockSpec((1, n, f_out), lambda i: (i, 0, 0)),
        out_shape=jax.ShapeDtypeStruct((b, n, f_out), jnp.float32),
        compiler_params=pltpu.CompilerParams(
            dimension_semantics=("parallel",)),
    )(h, W)

    a_row = a.reshape(1, 2 * f_out)
    adjT = jnp.swapaxes(adj, 1, 2)
    out = pl.pallas_call(
        _gat_kernel,
        grid=(b, n // _ROW_BLK),
        in_specs=[
            pl.BlockSpec((1, n, f_out), lambda i, j: (i, 0, 0)),
            pl.BlockSpec((1, n, _ROW_BLK), lambda i, j: (i, 0, j)),
            pl.BlockSpec((1, 2 * f_out), lambda i, j: (0, 0)),
        ],
        out_specs=pl.BlockSpec((1, _ROW_BLK, f_out), lambda i, j: (i, j, 0)),
        out_shape=jax.ShapeDtypeStruct((b, n, f_out), jnp.float32),
        scratch_shapes=[
            pltpu.VMEM((n, _ROW_BLK), jnp.float32),
        ],
        compiler_params=pltpu.CompilerParams(
            dimension_semantics=("parallel", "parallel")),
    )(wh, adjT, a_row)
    return out


# speedup vs baseline: 1.0486x; 1.0072x over previous
import jax
import jax.numpy as jnp
from jax.experimental import pallas as pl
from jax.experimental.pallas import tpu as pltpu

_K = 32
_ALPHA = 0.2
_NEG = -9e15
_ROW_BLK = 512
_ONE_BITS = 0x3F800001


def _wh_kernel(h_ref, w_ref, wh_ref):
    wh_ref[0] = jnp.dot(h_ref[0], w_ref[...], preferred_element_type=jnp.float32)


def _gat_kernel(wh_ref, adjT_ref, a_ref, out_ref, x_ref):
    f_out = wh_ref.shape[2]
    r = pl.program_id(1)
    wh = wh_ref[0]
    adjT = adjT_ref[0]
    a1 = a_ref[0:1, 0:f_out]
    a2 = a_ref[0:1, f_out:2 * f_out]
    wh_rows = wh_ref[0, pl.ds(r * _ROW_BLK, _ROW_BLK), :]
    f1c = jax.lax.dot_general(wh_rows, a1, (((1,), (1,)), ((), ())),
                              preferred_element_type=jnp.float32)
    f2r = jax.lax.dot_general(a2, wh, (((1,), (1,)), ((), ())),
                              preferred_element_type=jnp.float32)
    f1r = jnp.transpose(f1c)
    f2c = jnp.transpose(f2r)
    e = f2c + f1r
    e = jnp.maximum(e, _ALPHA * e)
    e = jnp.where(adjT > 0, e, _NEG)
    m = jnp.max(e, axis=0, keepdims=True)
    p = jnp.exp(e - m)
    s = jnp.sum(p, axis=0, keepdims=True)
    attn = p / s
    x_ref[...] = attn

    cols = attn.shape[1]
    lo0 = jnp.zeros((1, cols), jnp.int32)
    hi0 = jnp.full((1, cols), _ONE_BITS, jnp.int32)

    def body(it, state):
        lo, hi = state
        mid = jax.lax.shift_right_logical(lo + hi, 1)
        t = jax.lax.bitcast_convert_type(mid, jnp.float32)
        cf = jnp.sum(jnp.where(x_ref[...] >= t, 1.0, 0.0), axis=0,
                     keepdims=True)
        gek = cf >= _K
        lo = jnp.where(gek, mid, lo)
        hi = jnp.where(gek, hi, mid)
        return lo, hi

    lo, _ = jax.lax.fori_loop(0, 30, body, (lo0, hi0))
    thresh = jax.lax.bitcast_convert_type(lo, jnp.float32)

    x = x_ref[...]
    xm = jnp.where(x < thresh, 0.0, x)
    hp = jax.lax.dot_general(xm, wh, (((0,), (0,)), ((), ())),
                             preferred_element_type=jnp.float32)
    out_ref[0] = jnp.where(hp > 0, hp, jnp.exp(hp) - 1.0)


def kernel(h, adj, W, a):
    b, n, f_in = h.shape
    f_out = W.shape[1]
    wh = pl.pallas_call(
        _wh_kernel,
        grid=(b,),
        in_specs=[
            pl.BlockSpec((1, n, f_in), lambda i: (i, 0, 0)),
            pl.BlockSpec((f_in, f_out), lambda i: (0, 0)),
        ],
        out_specs=pl.BlockSpec((1, n, f_out), lambda i: (i, 0, 0)),
        out_shape=jax.ShapeDtypeStruct((b, n, f_out), jnp.float32),
        compiler_params=pltpu.CompilerParams(
            dimension_semantics=("parallel",)),
    )(h, W)

    a_row = a.reshape(1, 2 * f_out)
    adjT = jnp.swapaxes(adj, 1, 2)
    out = pl.pallas_call(
        _gat_kernel,
        grid=(b, n // _ROW_BLK),
        in_specs=[
            pl.BlockSpec((1, n, f_out), lambda i, j: (i, 0, 0)),
            pl.BlockSpec((1, n, _ROW_BLK), lambda i, j: (i, 0, j)),
            pl.BlockSpec((1, 2 * f_out), lambda i, j: (0, 0)),
        ],
        out_specs=pl.BlockSpec((1, _ROW_BLK, f_out), lambda i, j: (i, j, 0)),
        out_shape=jax.ShapeDtypeStruct((b, n, f_out), jnp.float32),
        scratch_shapes=[
            pltpu.VMEM((n, _ROW_BLK), jnp.float32),
        ],
        compiler_params=pltpu.CompilerParams(
            dimension_semantics=("parallel", "parallel")),
    )(wh, adjT, a_row)
    return out
